# baseline (speedup 1.0000x reference)
import functools

import jax
import jax.numpy as jnp
from jax.experimental import pallas as pl
from jax.experimental.pallas import tpu as pltpu


def _fused_kernel(nb, B, n, precision,
                  x_ref, A_ref, cb_ref, gnw_ref, gnb_ref, gnms_ref,
                  lw_ref, lb_ref, out_ref, halo_ref):
    j = pl.program_id(0)

    @pl.when(j == 0)
    def _init():
        halo_ref[...] = jnp.zeros_like(halo_ref)
        out_ref[...] = jnp.zeros_like(out_ref)

    feat = x_ref[...]
    is_last = j == nb - 1
    row_ids = jax.lax.broadcasted_iota(jnp.int32, (B, 1), 0)
    last_row = jnp.logical_and(row_ids == B - 1, is_last)

    x = feat
    for i in range(4):
        xb = x.astype(jnp.bfloat16)
        h = halo_ref[i, 0:3, :]
        halo_ref[i, 0:3, :] = xb[B - 3:B, :]
        ext = jnp.concatenate([h, xb], axis=0)
        zcat = jnp.concatenate(
            [xb, ext[2:B + 2], ext[1:B + 1], ext[0:B]], axis=1)
        A = A_ref[i]
        b = cb_ref[i][None, :]
        y = jax.lax.dot_general(
            zcat, A, (((1,), (0,)), ((), ())),
            preferred_element_type=jnp.float32, precision=precision) + b
        yfix = jax.lax.dot_general(
            xb[B - 1:B, :], A[0:128, :], (((1,), (0,)), ((), ())),
            preferred_element_type=jnp.float32, precision=precision) + b
        y = jnp.where(last_row, yfix, y)
        u = y * (1.0 - gnms_ref[i][None, :])
        g = gnw_ref[i][None, :] * (u * jax.lax.rsqrt(u * u + 1e-5)) \
            + gnb_ref[i][None, :]
        if i < 3:
            x = jnp.maximum(g, 0.1 * g)
        else:
            x = jnp.maximum(feat + g, 0.0)

    out_ref[...] += jnp.sum(x, axis=0, keepdims=True)

    @pl.when(is_last)
    def _finish():
        pooled = out_ref[...] * (1.0 / n)
        t = jax.lax.dot_general(
            pooled, lw_ref[...], (((1,), (1,)), ((), ())),
            preferred_element_type=jnp.float32,
            precision=jax.lax.Precision.HIGHEST) + lb_ref[...][None, :]
        out_ref[...] = jnp.maximum(t, 0.0) + jnp.log1p(jnp.exp(-jnp.abs(t)))


def _pick_block(n):
    for cand in (4000, 2000, 1000, 500, 200, 100, 40, 16, 8):
        if n % cand == 0:
            return cand
    return n


@jax.jit
def kernel(feat, conv_w, conv_b, gn_w, gn_b, gn_ms, lin_w, lin_b):
    n, d = feat.shape[1], feat.shape[2]
    x = feat.reshape(n, d)
    A = jnp.concatenate(
        [conv_w[:, 0] - conv_w[:, 2],
         3.0 * conv_w[:, 3] - conv_w[:, 1],
         2.0 * conv_w[:, 2],
         -4.0 * conv_w[:, 3]], axis=1).astype(jnp.bfloat16)

    B = _pick_block(n)
    nb = n // B
    full = lambda s: pl.BlockSpec(s, lambda j: (0,) * len(s))
    out = pl.pallas_call(
        functools.partial(_fused_kernel, nb, B, n,
                          jax.lax.Precision.DEFAULT),
        grid=(nb,),
        in_specs=[
            pl.BlockSpec((B, d), lambda j: (j, 0)),
            full((4, 4 * d, d)),
            full((4, d)),
            full((4, d)),
            full((4, d)),
            full((4, d)),
            full((d, d)),
            full((d,)),
        ],
        out_specs=pl.BlockSpec((1, d), lambda j: (0, 0)),
        out_shape=jax.ShapeDtypeStruct((1, d), jnp.float32),
        scratch_shapes=[pltpu.VMEM((4, 8, d), jnp.bfloat16)],
    )(x, A, conv_b, gn_w, gn_b, gn_ms, lin_w, lin_b)
    return out.reshape(d)


# speedup vs baseline: 1.1500x; 1.1500x over previous
import functools

import jax
import jax.numpy as jnp
from jax.experimental import pallas as pl
from jax.experimental.pallas import tpu as pltpu

_EPS = 1e-5


def _gn_act(u, w, b, is_final, feat):
    g = w * (u * jax.lax.rsqrt(u * u + _EPS)) + b
    if is_final:
        return jnp.maximum(feat + g, 0.0)
    return jnp.maximum(g, 0.1 * g)


def _fused_kernel(nb, B, n, x_ref, A_ref, cb_ref, gnw_ref, gnb_ref,
                  lw_ref, lb_ref, out_ref, halo_ref):
    j = pl.program_id(0)

    @pl.when(j == 0)
    def _init():
        halo_ref[...] = jnp.zeros_like(halo_ref)
        out_ref[...] = jnp.zeros_like(out_ref)

    feat = x_ref[...]

    x = feat
    for i in range(4):
        h = halo_ref[i, 0:3, :]
        halo_ref[i, 0:3, :] = x[B - 3:B, :]
        ext = jnp.concatenate([h, x], axis=0)
        zcat = jnp.concatenate(
            [x, ext[2:B + 2], ext[1:B + 1], ext[0:B]], axis=1)
        u = jax.lax.dot_general(
            zcat, A_ref[i], (((1,), (0,)), ((), ())),
            preferred_element_type=jnp.float32) + cb_ref[i][None, :]
        x = _gn_act(u, gnw_ref[i][None, :], gnb_ref[i][None, :], i == 3, feat)

    out_ref[...] += jnp.sum(x, axis=0, keepdims=True)

    @pl.when(j == nb - 1)
    def _finish():
        fl = feat[B - 1:B, :]
        v = fl
        for i in range(4):
            u = jax.lax.dot_general(
                v, A_ref[i][0:128, :], (((1,), (0,)), ((), ())),
                preferred_element_type=jnp.float32) + cb_ref[i][None, :]
            v = _gn_act(u, gnw_ref[i][None, :], gnb_ref[i][None, :],
                        i == 3, fl)
        pooled = (out_ref[...] + (v - x[B - 1:B, :])) * (1.0 / n)
        t = jax.lax.dot_general(
            pooled, lw_ref[...], (((1,), (1,)), ((), ())),
            preferred_element_type=jnp.float32,
            precision=jax.lax.Precision.HIGHEST) + lb_ref[...][None, :]
        out_ref[...] = jnp.maximum(t, 0.0) + jnp.log1p(jnp.exp(-jnp.abs(t)))


def _pick_block(n):
    for cand in (4000, 2000, 1000, 500, 200, 100, 40, 16, 8):
        if n % cand == 0:
            return cand
    return n


@jax.jit
def kernel(feat, conv_w, conv_b, gn_w, gn_b, gn_ms, lin_w, lin_b):
    n, d = feat.shape[1], feat.shape[2]
    x = feat.reshape(n, d)
    c = 1.0 - gn_ms
    A = jnp.concatenate(
        [conv_w[:, 0] - conv_w[:, 2],
         3.0 * conv_w[:, 3] - conv_w[:, 1],
         2.0 * conv_w[:, 2],
         -4.0 * conv_w[:, 3]], axis=1) * c[:, None, :]
    cb = conv_b * c

    B = _pick_block(n)
    nb = n // B
    full = lambda s: pl.BlockSpec(s, lambda j: (0,) * len(s))
    out = pl.pallas_call(
        functools.partial(_fused_kernel, nb, B, n),
        grid=(nb,),
        in_specs=[
            pl.BlockSpec((B, d), lambda j: (j, 0)),
            full((4, 4 * d, d)),
            full((4, d)),
            full((4, d)),
            full((4, d)),
            full((d, d)),
            full((d,)),
        ],
        out_specs=pl.BlockSpec((1, d), lambda j: (0, 0)),
        out_shape=jax.ShapeDtypeStruct((1, d), jnp.float32),
        scratch_shapes=[pltpu.VMEM((4, 8, d), jnp.float32)],
    )(x, A, cb, gn_w, gn_b, lin_w, lin_b)
    return out.reshape(d)


# speedup vs baseline: 1.1853x; 1.0307x over previous
import functools

import jax
import jax.numpy as jnp
from jax.experimental import pallas as pl
from jax.experimental.pallas import tpu as pltpu

_EPS = 1e-5


def _gn_act(u, w, b, is_final, feat):
    g = w * (u * jax.lax.rsqrt(u * u + _EPS)) + b
    if is_final:
        return jnp.maximum(feat + g, 0.0)
    return jnp.maximum(g, 0.1 * g)


def _fused_kernel(nb, B, n, x_ref, A_ref, cb_ref, gnw_ref, gnb_ref,
                  lw_ref, lb_ref, out_ref, halo_ref):
    j = pl.program_id(0)

    @pl.when(j == 0)
    def _init():
        halo_ref[...] = jnp.zeros_like(halo_ref)
        out_ref[...] = jnp.zeros_like(out_ref)

    feat = x_ref[...]

    x = feat
    for i in range(4):
        h = halo_ref[i, 0:3, :]
        halo_ref[i, 0:3, :] = x[B - 3:B, :]
        ext = jnp.concatenate([h, x], axis=0)
        zcat = jnp.concatenate(
            [x, ext[2:B + 2], ext[1:B + 1], ext[0:B]], axis=1)
        u = jax.lax.dot_general(
            zcat, A_ref[i], (((1,), (0,)), ((), ())),
            preferred_element_type=jnp.float32) + cb_ref[i][None, :]
        x = _gn_act(u, gnw_ref[i][None, :], gnb_ref[i][None, :], i == 3, feat)

    out_ref[...] += jnp.sum(x, axis=0, keepdims=True)

    @pl.when(j == nb - 1)
    def _finish():
        fl = feat[B - 1:B, :]
        v = fl
        for i in range(4):
            u = jax.lax.dot_general(
                v, A_ref[i][0:128, :], (((1,), (0,)), ((), ())),
                preferred_element_type=jnp.float32) + cb_ref[i][None, :]
            v = _gn_act(u, gnw_ref[i][None, :], gnb_ref[i][None, :],
                        i == 3, fl)
        pooled = (out_ref[...] + (v - x[B - 1:B, :])) * (1.0 / n)
        t = jax.lax.dot_general(
            pooled, lw_ref[...], (((1,), (1,)), ((), ())),
            preferred_element_type=jnp.float32,
            precision=jax.lax.Precision.HIGHEST) + lb_ref[...][None, :]
        out_ref[...] = jnp.maximum(t, 0.0) + jnp.log1p(jnp.exp(-jnp.abs(t)))


def _pick_block(n):
    for cand in (5000, 4000, 2000, 1000, 500, 200, 100, 40, 16, 8):
        if n % cand == 0:
            return cand
    return n


@jax.jit
def kernel(feat, conv_w, conv_b, gn_w, gn_b, gn_ms, lin_w, lin_b):
    n, d = feat.shape[1], feat.shape[2]
    x = feat.reshape(n, d)
    c = 1.0 - gn_ms
    A = jnp.concatenate(
        [conv_w[:, 0] - conv_w[:, 2],
         3.0 * conv_w[:, 3] - conv_w[:, 1],
         2.0 * conv_w[:, 2],
         -4.0 * conv_w[:, 3]], axis=1) * c[:, None, :]
    cb = conv_b * c

    B = _pick_block(n)
    nb = n // B
    full = lambda s: pl.BlockSpec(s, lambda j: (0,) * len(s))
    out = pl.pallas_call(
        functools.partial(_fused_kernel, nb, B, n),
        grid=(nb,),
        in_specs=[
            pl.BlockSpec((B, d), lambda j: (j, 0)),
            full((4, 4 * d, d)),
            full((4, d)),
            full((4, d)),
            full((4, d)),
            full((d, d)),
            full((d,)),
        ],
        out_specs=pl.BlockSpec((1, d), lambda j: (0, 0)),
        out_shape=jax.ShapeDtypeStruct((1, d), jnp.float32),
        scratch_shapes=[pltpu.VMEM((4, 8, d), jnp.float32)],
    )(x, A, cb, gn_w, gn_b, lin_w, lin_b)
    return out.reshape(d)


# speedup vs baseline: 1.2393x; 1.0455x over previous
import functools

import jax
import jax.numpy as jnp
from jax.experimental import pallas as pl
from jax.experimental.pallas import tpu as pltpu

_EPS = 1e-5


def _gn_act(u, w, b, is_final, feat):
    g = w * (u * jax.lax.rsqrt(u * u + _EPS)) + b
    if is_final:
        return jnp.maximum(feat + g, 0.0)
    return jnp.maximum(g, 0.1 * g)


def _fused_kernel(nb, B, n, x_ref, A_ref, cb_ref, gnw_ref, gnb_ref,
                  lw_ref, lb_ref, out_ref, halo_ref):
    j = pl.program_id(0)

    @pl.when(j == 0)
    def _init():
        halo_ref[...] = jnp.zeros_like(halo_ref)
        out_ref[...] = jnp.zeros_like(out_ref)

    feat = x_ref[...]

    x = feat
    for i in range(4):
        h = halo_ref[i, 0:3, :]
        halo_ref[i, 0:3, :] = x[B - 3:B, :]
        ext = jnp.concatenate([h, x], axis=0)
        zcat = jnp.concatenate(
            [x, ext[2:B + 2], ext[1:B + 1], ext[0:B]], axis=1)
        u = jax.lax.dot_general(
            zcat, A_ref[i], (((1,), (0,)), ((), ())),
            preferred_element_type=jnp.float32) + cb_ref[i][None, :]
        x = _gn_act(u, gnw_ref[i][None, :], gnb_ref[i][None, :], i == 3, feat)

    out_ref[...] += jnp.sum(x, axis=0, keepdims=True)

    @pl.when(j == nb - 1)
    def _finish():
        fl = feat[B - 1:B, :]
        v = fl
        for i in range(4):
            u = jax.lax.dot_general(
                v, A_ref[i][0:128, :], (((1,), (0,)), ((), ())),
                preferred_element_type=jnp.float32) + cb_ref[i][None, :]
            v = _gn_act(u, gnw_ref[i][None, :], gnb_ref[i][None, :],
                        i == 3, fl)
        pooled = (out_ref[...] + (v - x[B - 1:B, :])) * (1.0 / n)
        t = jax.lax.dot_general(
            pooled, lw_ref[...], (((1,), (1,)), ((), ())),
            preferred_element_type=jnp.float32,
            precision=jax.lax.Precision.HIGHEST) + lb_ref[...][None, :]
        out_ref[...] = jnp.maximum(t, 0.0) + jnp.log1p(jnp.exp(-jnp.abs(t)))


def _pick_block(n):
    for cand in (10000, 5000, 4000, 2000, 1000, 500, 200, 100, 40, 16, 8):
        if n % cand == 0:
            return cand
    return n


@jax.jit
def kernel(feat, conv_w, conv_b, gn_w, gn_b, gn_ms, lin_w, lin_b):
    n, d = feat.shape[1], feat.shape[2]
    x = feat.reshape(n, d)
    c = 1.0 - gn_ms
    A = jnp.concatenate(
        [conv_w[:, 0] - conv_w[:, 2],
         3.0 * conv_w[:, 3] - conv_w[:, 1],
         2.0 * conv_w[:, 2],
         -4.0 * conv_w[:, 3]], axis=1) * c[:, None, :]
    cb = conv_b * c

    B = _pick_block(n)
    nb = n // B
    full = lambda s: pl.BlockSpec(s, lambda j: (0,) * len(s))
    out = pl.pallas_call(
        functools.partial(_fused_kernel, nb, B, n),
        grid=(nb,),
        in_specs=[
            pl.BlockSpec((B, d), lambda j: (j, 0)),
            full((4, 4 * d, d)),
            full((4, d)),
            full((4, d)),
            full((4, d)),
            full((d, d)),
            full((d,)),
        ],
        out_specs=pl.BlockSpec((1, d), lambda j: (0, 0)),
        out_shape=jax.ShapeDtypeStruct((1, d), jnp.float32),
        scratch_shapes=[pltpu.VMEM((4, 8, d), jnp.float32)],
    )(x, A, cb, gn_w, gn_b, lin_w, lin_b)
    return out.reshape(d)


# speedup vs baseline: 1.2568x; 1.0141x over previous
import functools

import jax
import jax.numpy as jnp
from jax.experimental import pallas as pl
from jax.experimental.pallas import tpu as pltpu

_EPS = 1e-5


def _gn_act(u, w, b, is_final, feat):
    g = w * (u * jax.lax.rsqrt(u * u + _EPS)) + b
    if is_final:
        return jnp.maximum(feat + g, 0.0)
    return jnp.maximum(g, 0.1 * g)


def _fused_kernel(nb, B, n, x_ref, A_ref, cb_ref, gnw_ref, gnb_ref,
                  lw_ref, lb_ref, out_ref, halo_ref):
    j = pl.program_id(0)

    @pl.when(j == 0)
    def _init():
        halo_ref[...] = jnp.zeros_like(halo_ref)
        out_ref[...] = jnp.zeros_like(out_ref)

    feat = x_ref[...]

    x = feat
    for i in range(4):
        h = halo_ref[i, 0:3, :]
        halo_ref[i, 0:3, :] = x[B - 3:B, :]
        ext = jnp.concatenate([h, x], axis=0)
        zcat = jnp.concatenate(
            [x, ext[2:B + 2], ext[1:B + 1], ext[0:B]], axis=1)
        u = jax.lax.dot_general(
            zcat, A_ref[i], (((1,), (0,)), ((), ())),
            preferred_element_type=jnp.float32) + cb_ref[i][None, :]
        x = _gn_act(u, gnw_ref[i][None, :], gnb_ref[i][None, :], i == 3, feat)

    out_ref[...] += jnp.sum(x, axis=0, keepdims=True)

    @pl.when(j == nb - 1)
    def _finish():
        fl = feat[B - 1:B, :]
        v = fl
        for i in range(4):
            u = jax.lax.dot_general(
                v, A_ref[i][0:128, :], (((1,), (0,)), ((), ())),
                preferred_element_type=jnp.float32) + cb_ref[i][None, :]
            v = _gn_act(u, gnw_ref[i][None, :], gnb_ref[i][None, :],
                        i == 3, fl)
        pooled = (out_ref[...] + (v - x[B - 1:B, :])) * (1.0 / n)
        t = jax.lax.dot_general(
            pooled, lw_ref[...], (((1,), (1,)), ((), ())),
            preferred_element_type=jnp.float32,
            precision=jax.lax.Precision.HIGHEST) + lb_ref[...][None, :]
        out_ref[...] = jnp.maximum(t, 0.0) + jnp.log1p(jnp.exp(-jnp.abs(t)))


def _pick_block(n):
    for cand in (20000, 10000, 5000, 4000, 2000, 1000, 500, 200, 100, 40, 16, 8):
        if n % cand == 0:
            return cand
    return n


@jax.jit
def kernel(feat, conv_w, conv_b, gn_w, gn_b, gn_ms, lin_w, lin_b):
    n, d = feat.shape[1], feat.shape[2]
    x = feat.reshape(n, d)
    c = 1.0 - gn_ms
    A = jnp.concatenate(
        [conv_w[:, 0] - conv_w[:, 2],
         3.0 * conv_w[:, 3] - conv_w[:, 1],
         2.0 * conv_w[:, 2],
         -4.0 * conv_w[:, 3]], axis=1) * c[:, None, :]
    cb = conv_b * c

    B = _pick_block(n)
    nb = n // B
    full = lambda s: pl.BlockSpec(s, lambda j: (0,) * len(s))
    out = pl.pallas_call(
        functools.partial(_fused_kernel, nb, B, n),
        grid=(nb,),
        in_specs=[
            pl.BlockSpec((B, d), lambda j: (j, 0)),
            full((4, 4 * d, d)),
            full((4, d)),
            full((4, d)),
            full((4, d)),
            full((d, d)),
            full((d,)),
        ],
        out_specs=pl.BlockSpec((1, d), lambda j: (0, 0)),
        out_shape=jax.ShapeDtypeStruct((1, d), jnp.float32),
        scratch_shapes=[pltpu.VMEM((4, 8, d), jnp.float32)],
    )(x, A, cb, gn_w, gn_b, lin_w, lin_b)
    return out.reshape(d)
